# baseline (device time: 117902 ns/iter reference)
import jax
import jax.numpy as jnp
from jax import lax
from jax.experimental import pallas as pl
from jax.experimental.pallas import tpu as pltpu

N_DEV = 4
HPD = 8
SQ = 2048
SKV = 2048
DH = 128
DIN = 1024
DSH = HPD * DH
BQ = 512
SCALE = 0.08838834764831843
WINDOW = 128
NGLOB = 32
CH = SQ // N_DEV
HC = DIN // 2
SB = HC // 2
GB = 128
HQ = 256
HBW = HQ + 2 * WINDOW

OFF = (0, 3, 1, 2)

MESH = pl.DeviceIdType.MESH


def _gc_of(qc, i):
    off = jnp.where(qc == 1, 3, jnp.where(qc == 2, 1, jnp.where(qc == 3, 2, 0)))
    return lax.rem(i + off, N_DEV)


def _body(i_ref, x_ref, wq_ref, k_hbm, v_hbm, wo_ref, out_ref, ctx_sc, q_sc,
          k_buf, v_buf, k_sem, v_sem,
          cw_comm, ccw_comm,
          cw_send, cw_recv, ccw_send, ccw_recv,
          agcw_send, agcw_recv, agccw_send, agccw_recv):
    qc = pl.program_id(0)
    h = pl.program_id(1)
    i = i_ref[0]
    left = lax.rem(i + N_DEV - 1, N_DEV)
    right = lax.rem(i + 1, N_DEV)
    gc = _gc_of(qc, i)
    t = qc * HPD + h
    slot = lax.rem(t, 2)

    @pl.when((qc == 0) & (h == 0))
    def _barrier():
        barrier = pltpu.get_barrier_semaphore()
        for nbr in (left, right):
            pl.semaphore_signal(barrier, inc=1, device_id=(nbr,),
                                device_id_type=MESH)
        pl.semaphore_wait(barrier, 2)

    def _kv_dma(sl, hd):
        k_c = pltpu.make_async_copy(
            k_hbm.at[:, hd, :], k_buf.at[sl], k_sem.at[sl]
        )
        v_c = pltpu.make_async_copy(
            v_hbm.at[:, hd, :], v_buf.at[sl], v_sem.at[sl]
        )
        return k_c, v_c

    @pl.when(t == 0)
    def _prologue():
        for sl in (0, 1):
            k_c, v_c = _kv_dma(sl, i * HPD + sl)
            k_c.start()
            v_c.start()

    @pl.when((t > 0) & (t < N_DEV * HPD - 1))
    def _prefetch_next():
        nt = t + 1
        k_c, v_c = _kv_dma(lax.rem(nt, 2), i * HPD + lax.rem(nt, HPD))
        k_c.start()
        v_c.start()

    k_c, v_c = _kv_dma(slot, i * HPD + h)
    k_c.wait()
    v_c.wait()
    k_col = k_buf.at[slot]
    v_col = v_buf.at[slot]

    @pl.when(h == 0)
    def _qproj():
        q_sc[...] = jnp.dot(
            x_ref[...], wq_ref[...], preferred_element_type=jnp.float32
        )

    q = q_sc[:, pl.ds(h * DH, DH)]


    def _banded(half, qh):
        row0 = gc * BQ + half * HQ
        bs = jnp.minimum(row0 - WINDOW, SKV - HBW)
        off = row0 - bs
        kg = k_col[:GB, :]
        kb = k_col[pl.ds(bs, HBW), :]
        s_g = lax.dot_general(
            qh, kg, (((1,), (1,)), ((), ())),
            preferred_element_type=jnp.float32,
        ) * SCALE
        s_b = lax.dot_general(
            qh, kb, (((1,), (1,)), ((), ())),
            preferred_element_type=jnp.float32,
        ) * SCALE
        kig = lax.broadcasted_iota(jnp.int32, (HQ, GB), 1)
        s_g = jnp.where(kig < NGLOB, s_g, -1e9)
        r = lax.broadcasted_iota(jnp.int32, (HQ, HBW), 0)
        j = lax.broadcasted_iota(jnp.int32, (HQ, HBW), 1)
        s_b = jnp.where(
            (j >= r + off - WINDOW) & (j <= r + off + WINDOW), s_b, -1e9
        )
        w_g = jnp.exp(s_g)
        w_b = jnp.exp(s_b)
        denom = (
            jnp.sum(w_g, axis=1, keepdims=True)
            + jnp.sum(w_b, axis=1, keepdims=True)
        )
        ctx = jnp.dot(w_g, v_col[:GB, :], preferred_element_type=jnp.float32)
        ctx += jnp.dot(
            w_b, v_col[pl.ds(bs, HBW), :], preferred_element_type=jnp.float32
        )
        ctx_sc[pl.ds(half * HQ, HQ), pl.ds(h * DH, DH)] = ctx / denom

    qh0 = q[:HQ]

    @pl.when(gc == 0)
    def _dense0():
        kb = k_col[:HBW, :]
        s1 = lax.dot_general(
            qh0, kb, (((1,), (1,)), ((), ())),
            preferred_element_type=jnp.float32,
        ) * SCALE
        qi = lax.broadcasted_iota(jnp.int32, (HQ, HBW), 0)
        ki = lax.broadcasted_iota(jnp.int32, (HQ, HBW), 1)
        mask = (jnp.abs(qi - ki) <= WINDOW) | (ki < NGLOB) | (qi < NGLOB)
        w1 = jnp.exp(jnp.where(mask, s1, -1e9))
        d1 = jnp.sum(w1, axis=1, keepdims=True)
        ctx = jnp.dot(w1, v_col[:HBW, :], preferred_element_type=jnp.float32)
        s2 = lax.dot_general(
            qh0[:NGLOB], k_col[HBW:, :], (((1,), (1,)), ((), ())),
            preferred_element_type=jnp.float32,
        ) * SCALE
        w2 = jnp.exp(s2)
        top = ctx[:NGLOB] + jnp.dot(
            w2, v_col[HBW:, :], preferred_element_type=jnp.float32
        )
        top_d = d1[:NGLOB] + jnp.sum(w2, axis=1, keepdims=True)
        ctx_sc[:HQ, pl.ds(h * DH, DH)] = jnp.concatenate(
            [top / top_d, ctx[NGLOB:] / d1[NGLOB:]], axis=0
        )

    @pl.when(gc > 0)
    def _banded0():
        _banded(0, qh0)

    _banded(1, q[HQ:])

    def _rs(s, cw_c, ccw_c):
        cw = pltpu.make_async_remote_copy(
            src_ref=out_ref.at[pl.ds(cw_c * CH, CH), pl.ds(0, HC)],
            dst_ref=cw_comm.at[s],
            send_sem=cw_send.at[s, 0], recv_sem=cw_recv.at[s, 0],
            device_id=(right,), device_id_type=MESH,
        )
        ccw = pltpu.make_async_remote_copy(
            src_ref=out_ref.at[pl.ds(ccw_c * CH, CH), pl.ds(HC, HC)],
            dst_ref=ccw_comm.at[s],
            send_sem=ccw_send.at[s, 0], recv_sem=ccw_recv.at[s, 0],
            device_id=(left,), device_id_type=MESH,
        )
        return cw, ccw

    def _rs_accum(s, cw_c, ccw_c):
        cw, ccw = _rs(s, cw_c, ccw_c)
        cw.wait()
        ccw.wait()
        acc = out_ref[pl.ds(cw_c * CH, CH), pl.ds(0, HC)] + cw_comm[s]
        out_ref[pl.ds(cw_c * CH, CH), pl.ds(0, HC)] = acc
        acc = out_ref[pl.ds(ccw_c * CH, CH), pl.ds(HC, HC)] + ccw_comm[s]
        out_ref[pl.ds(ccw_c * CH, CH), pl.ds(HC, HC)] = acc

    @pl.when(h == HPD - 1)
    def _chunk_done():
        partial = jnp.dot(
            ctx_sc[...], wo_ref[...], preferred_element_type=jnp.float32
        )
        out_ref[pl.ds(gc * CH, CH), :] = partial

        @pl.when(qc == 0)
        def _():
            cw, ccw = _rs(0, i, i)
            cw.start()
            ccw.start()

        @pl.when(qc == 2)
        def _():
            _rs_accum(0, left, right)
            cw, ccw = _rs(1, left, right)
            cw.start()
            ccw.start()

        @pl.when(qc == 3)
        def _():
            far = lax.rem(i + 2, N_DEV)
            _rs_accum(1, far, far)

            def _rs2(sub):
                cw = pltpu.make_async_remote_copy(
                    src_ref=out_ref.at[pl.ds(far * CH, CH),
                                       pl.ds(sub * SB, SB)],
                    dst_ref=cw_comm.at[2, :, pl.ds(sub * SB, SB)],
                    send_sem=cw_send.at[2, sub], recv_sem=cw_recv.at[2, sub],
                    device_id=(right,), device_id_type=MESH,
                )
                ccw = pltpu.make_async_remote_copy(
                    src_ref=out_ref.at[pl.ds(far * CH, CH),
                                       pl.ds(HC + sub * SB, SB)],
                    dst_ref=ccw_comm.at[2, :, pl.ds(sub * SB, SB)],
                    send_sem=ccw_send.at[2, sub], recv_sem=ccw_recv.at[2, sub],
                    device_id=(left,), device_id_type=MESH,
                )
                return cw, ccw

            def _ag(s, sub):
                cw_c = lax.rem(i + 1 - s + N_DEV, N_DEV)
                ccw_c = lax.rem(i - 1 + s + N_DEV, N_DEV)
                cw = pltpu.make_async_remote_copy(
                    src_ref=out_ref.at[pl.ds(cw_c * CH, CH),
                                       pl.ds(sub * SB, SB)],
                    dst_ref=out_ref.at[pl.ds(cw_c * CH, CH),
                                       pl.ds(sub * SB, SB)],
                    send_sem=agcw_send.at[s, sub],
                    recv_sem=agcw_recv.at[s, sub],
                    device_id=(right,), device_id_type=MESH,
                )
                ccw = pltpu.make_async_remote_copy(
                    src_ref=out_ref.at[pl.ds(ccw_c * CH, CH),
                                       pl.ds(HC + sub * SB, SB)],
                    dst_ref=out_ref.at[pl.ds(ccw_c * CH, CH),
                                       pl.ds(HC + sub * SB, SB)],
                    send_sem=agccw_send.at[s, sub],
                    recv_sem=agccw_recv.at[s, sub],
                    device_id=(left,), device_id_type=MESH,
                )
                return cw, ccw

            for sub in (0, 1):
                cw, ccw = _rs2(sub)
                cw.start()
                ccw.start()
            for sub in (0, 1):
                cw, ccw = _rs2(sub)
                cw.wait()
                ccw.wait()
                cs = pl.ds(sub * SB, SB)
                acc = out_ref[pl.ds(right * CH, CH), cs] \
                    + cw_comm[2, :, pl.ds(sub * SB, SB)]
                out_ref[pl.ds(right * CH, CH), cs] = acc
                ccs = pl.ds(HC + sub * SB, SB)
                acc = out_ref[pl.ds(left * CH, CH), ccs] \
                    + ccw_comm[2, :, pl.ds(sub * SB, SB)]
                out_ref[pl.ds(left * CH, CH), ccs] = acc
                cw, ccw = _ag(0, sub)
                cw.start()
                ccw.start()
            for s in (1, 2):
                for sub in (0, 1):
                    cw, ccw = _ag(s - 1, sub)
                    cw.wait()
                    ccw.wait()
                    cw, ccw = _ag(s, sub)
                    cw.start()
                    ccw.start()
            for sub in (0, 1):
                cw, ccw = _ag(2, sub)
                cw.wait()
                ccw.wait()


def kernel(x, Wq, K_ext, V_ext, Wo):
    i = lax.axis_index("i")
    x2 = x.reshape(SQ, DIN)
    K2 = K_ext.reshape(SKV, 32, DH)
    V2 = V_ext.reshape(SKV, 32, DH)
    iarr = jnp.full((1,), i, jnp.int32)

    grid_spec = pltpu.PrefetchScalarGridSpec(
        num_scalar_prefetch=1,
        grid=(N_DEV, HPD),
        in_specs=[
            pl.BlockSpec((BQ, DIN), lambda qc, h, ir: (_gc_of(qc, ir[0]), 0)),
            pl.BlockSpec((DIN, DSH), lambda qc, h, ir: (0, 0)),
            pl.BlockSpec(memory_space=pl.ANY),
            pl.BlockSpec(memory_space=pl.ANY),
            pl.BlockSpec((DSH, DIN), lambda qc, h, ir: (0, 0)),
        ],
        out_specs=pl.BlockSpec((SQ, DIN), lambda qc, h, ir: (0, 0)),
        scratch_shapes=[
            pltpu.VMEM((BQ, DSH), jnp.float32),
            pltpu.VMEM((BQ, DSH), jnp.float32),
            pltpu.VMEM((2, SKV, DH), jnp.float32),
            pltpu.VMEM((2, SKV, DH), jnp.float32),
            pltpu.SemaphoreType.DMA((2,)),
            pltpu.SemaphoreType.DMA((2,)),
            pltpu.VMEM((N_DEV - 1, CH, HC), jnp.float32),
            pltpu.VMEM((N_DEV - 1, CH, HC), jnp.float32),
        ] + [pltpu.SemaphoreType.DMA((N_DEV - 1, 2))] * 8,
    )
    out = pl.pallas_call(
        _body,
        grid_spec=grid_spec,
        out_shape=jax.ShapeDtypeStruct((SQ, DIN), jnp.float32),
        compiler_params=pltpu.CompilerParams(
            dimension_semantics=("arbitrary", "arbitrary"),
            collective_id=0,
        ),
    )(iarr, x2, Wq, K2, V2, Wo)
    return out.reshape(1, SQ, DIN)


# device time: 101562 ns/iter; 1.1609x vs baseline; 1.1609x over previous
import jax
import jax.numpy as jnp
from jax import lax
from jax.experimental import pallas as pl
from jax.experimental.pallas import tpu as pltpu

N_DEV = 4
HPD = 8
SQ = 2048
SKV = 2048
DH = 128
DIN = 1024
DSH = HPD * DH
BQ = 512
SCALE = 0.08838834764831843
WINDOW = 128
NGLOB = 32
CH = SQ // N_DEV
HC = DIN // 2
NSUB = 4
SB = HC // NSUB
GB = 128
BW = BQ + 2 * WINDOW

OFF = (0, 3, 1, 2)

MESH = pl.DeviceIdType.MESH


def _gc_of(qc, i):
    off = jnp.where(qc == 1, 3, jnp.where(qc == 2, 1, jnp.where(qc == 3, 2, 0)))
    return lax.rem(i + off, N_DEV)


def _body(i_ref, x_ref, wq_ref, k_hbm, v_hbm, wo_ref, out_ref, ctx_sc,
          k_buf, v_buf, kv_sem,
          cw_comm, ccw_comm,
          cw_send, cw_recv, ccw_send, ccw_recv,
          agcw_send, agcw_recv, agccw_send, agccw_recv):
    qc = pl.program_id(0)
    i = i_ref[0]
    left = lax.rem(i + N_DEV - 1, N_DEV)
    right = lax.rem(i + 1, N_DEV)
    gc = _gc_of(qc, i)

    @pl.when(qc == 0)
    def _barrier():
        barrier = pltpu.get_barrier_semaphore()
        for nbr in (left, right):
            pl.semaphore_signal(barrier, inc=1, device_id=(nbr,),
                                device_id_type=MESH)
        pl.semaphore_wait(barrier, 2)

    def _kv_dma(h):
        k_c = pltpu.make_async_copy(
            k_hbm.at[:, i * HPD + h, :], k_buf.at[h], kv_sem.at[0, h]
        )
        v_c = pltpu.make_async_copy(
            v_hbm.at[:, i * HPD + h, :], v_buf.at[h], kv_sem.at[1, h]
        )
        return k_c, v_c

    @pl.when(qc == 0)
    def _fetch_kv():
        for h in range(HPD):
            k_c, v_c = _kv_dma(h)
            k_c.start()
            v_c.start()

    q_all = jnp.dot(x_ref[...], wq_ref[...], preferred_element_type=jnp.float32)

    for h in range(HPD):
        @pl.when(qc == 0)
        def _wait_kv(h=h):
            k_c, v_c = _kv_dma(h)
            k_c.wait()
            v_c.wait()

        q = q_all[:, h * DH:(h + 1) * DH]
        k_col = k_buf.at[h]
        v_col = v_buf.at[h]

        @pl.when(gc == 0)
        def _dense(q=q, k_col=k_col, v_col=v_col, h=h):
            kb = k_col[:BW, :]
            s1 = lax.dot_general(
                q, kb, (((1,), (1,)), ((), ())),
                preferred_element_type=jnp.float32,
            ) * SCALE
            qi = lax.broadcasted_iota(jnp.int32, (BQ, BW), 0)
            ki = lax.broadcasted_iota(jnp.int32, (BQ, BW), 1)
            mask = (jnp.abs(qi - ki) <= WINDOW) | (ki < NGLOB) | (qi < NGLOB)
            w1 = jnp.exp(jnp.where(mask, s1, -1e9))
            d1 = jnp.sum(w1, axis=1, keepdims=True)
            ctx = jnp.dot(
                w1, v_col[:BW, :], preferred_element_type=jnp.float32
            )
            s2 = lax.dot_general(
                q[:NGLOB], k_col[BW:, :], (((1,), (1,)), ((), ())),
                preferred_element_type=jnp.float32,
            ) * SCALE
            w2 = jnp.exp(s2)
            top = ctx[:NGLOB] + jnp.dot(
                w2, v_col[BW:, :], preferred_element_type=jnp.float32
            )
            top_d = d1[:NGLOB] + jnp.sum(w2, axis=1, keepdims=True)
            ctx_sc[:, h * DH:(h + 1) * DH] = jnp.concatenate(
                [top / top_d, ctx[NGLOB:] / d1[NGLOB:]], axis=0
            )

        @pl.when(gc > 0)
        def _banded(q=q, k_col=k_col, v_col=v_col, h=h):
            bs = jnp.minimum(gc * BQ - WINDOW, SKV - BW)
            off = gc * BQ - bs
            kg = k_col[:GB, :]
            kb = k_col[pl.ds(bs, BW), :]
            s_g = lax.dot_general(
                q, kg, (((1,), (1,)), ((), ())),
                preferred_element_type=jnp.float32,
            ) * SCALE
            s_b = lax.dot_general(
                q, kb, (((1,), (1,)), ((), ())),
                preferred_element_type=jnp.float32,
            ) * SCALE
            kig = lax.broadcasted_iota(jnp.int32, (BQ, GB), 1)
            s_g = jnp.where(kig < NGLOB, s_g, -1e9)
            r = lax.broadcasted_iota(jnp.int32, (BQ, BW), 0)
            j = lax.broadcasted_iota(jnp.int32, (BQ, BW), 1)
            s_b = jnp.where(
                (j >= r + off - WINDOW) & (j <= r + off + WINDOW), s_b, -1e9
            )
            w_g = jnp.exp(s_g)
            w_b = jnp.exp(s_b)
            denom = (
                jnp.sum(w_g, axis=1, keepdims=True)
                + jnp.sum(w_b, axis=1, keepdims=True)
            )
            ctx = jnp.dot(
                w_g, v_col[:GB, :], preferred_element_type=jnp.float32
            )
            ctx += jnp.dot(
                w_b, v_col[pl.ds(bs, BW), :],
                preferred_element_type=jnp.float32,
            )
            ctx_sc[:, h * DH:(h + 1) * DH] = ctx / denom

    partial = jnp.dot(
        ctx_sc[...], wo_ref[...], preferred_element_type=jnp.float32
    )
    out_ref[pl.ds(gc * CH, CH), :] = partial

    def _rs(s, cw_c, ccw_c):
        cw = pltpu.make_async_remote_copy(
            src_ref=out_ref.at[pl.ds(cw_c * CH, CH), pl.ds(0, HC)],
            dst_ref=cw_comm.at[s],
            send_sem=cw_send.at[s, 0], recv_sem=cw_recv.at[s, 0],
            device_id=(right,), device_id_type=MESH,
        )
        ccw = pltpu.make_async_remote_copy(
            src_ref=out_ref.at[pl.ds(ccw_c * CH, CH), pl.ds(HC, HC)],
            dst_ref=ccw_comm.at[s],
            send_sem=ccw_send.at[s, 0], recv_sem=ccw_recv.at[s, 0],
            device_id=(left,), device_id_type=MESH,
        )
        return cw, ccw

    def _rs_accum(s, cw_c, ccw_c):
        cw, ccw = _rs(s, cw_c, ccw_c)
        cw.wait()
        ccw.wait()
        acc = out_ref[pl.ds(cw_c * CH, CH), pl.ds(0, HC)] + cw_comm[s]
        out_ref[pl.ds(cw_c * CH, CH), pl.ds(0, HC)] = acc
        acc = out_ref[pl.ds(ccw_c * CH, CH), pl.ds(HC, HC)] + ccw_comm[s]
        out_ref[pl.ds(ccw_c * CH, CH), pl.ds(HC, HC)] = acc

    @pl.when(qc == 0)
    def _comm0():
        cw, ccw = _rs(0, i, i)
        cw.start()
        ccw.start()

    @pl.when(qc == 2)
    def _comm2():
        _rs_accum(0, left, right)
        cw, ccw = _rs(1, left, right)
        cw.start()
        ccw.start()

    @pl.when(qc == 3)
    def _comm3():
        far = lax.rem(i + 2, N_DEV)
        _rs_accum(1, far, far)

        def _rs2(sub):
            cw = pltpu.make_async_remote_copy(
                src_ref=out_ref.at[pl.ds(far * CH, CH),
                                   pl.ds(sub * SB, SB)],
                dst_ref=cw_comm.at[2, :, pl.ds(sub * SB, SB)],
                send_sem=cw_send.at[2, sub], recv_sem=cw_recv.at[2, sub],
                device_id=(right,), device_id_type=MESH,
            )
            ccw = pltpu.make_async_remote_copy(
                src_ref=out_ref.at[pl.ds(far * CH, CH),
                                   pl.ds(HC + sub * SB, SB)],
                dst_ref=ccw_comm.at[2, :, pl.ds(sub * SB, SB)],
                send_sem=ccw_send.at[2, sub], recv_sem=ccw_recv.at[2, sub],
                device_id=(left,), device_id_type=MESH,
            )
            return cw, ccw

        def _ag(s, sub):
            cw_c = lax.rem(i + 1 - s + N_DEV, N_DEV)
            ccw_c = lax.rem(i - 1 + s + N_DEV, N_DEV)
            cw = pltpu.make_async_remote_copy(
                src_ref=out_ref.at[pl.ds(cw_c * CH, CH),
                                   pl.ds(sub * SB, SB)],
                dst_ref=out_ref.at[pl.ds(cw_c * CH, CH),
                                   pl.ds(sub * SB, SB)],
                send_sem=agcw_send.at[s, sub],
                recv_sem=agcw_recv.at[s, sub],
                device_id=(right,), device_id_type=MESH,
            )
            ccw = pltpu.make_async_remote_copy(
                src_ref=out_ref.at[pl.ds(ccw_c * CH, CH),
                                   pl.ds(HC + sub * SB, SB)],
                dst_ref=out_ref.at[pl.ds(ccw_c * CH, CH),
                                   pl.ds(HC + sub * SB, SB)],
                send_sem=agccw_send.at[s, sub],
                recv_sem=agccw_recv.at[s, sub],
                device_id=(left,), device_id_type=MESH,
            )
            return cw, ccw

        for sub in range(NSUB):
            cw, ccw = _rs2(sub)
            cw.start()
            ccw.start()
        for sub in range(NSUB):
            cw, ccw = _rs2(sub)
            cw.wait()
            ccw.wait()
            cs = pl.ds(sub * SB, SB)
            acc = out_ref[pl.ds(right * CH, CH), cs] \
                + cw_comm[2, :, pl.ds(sub * SB, SB)]
            out_ref[pl.ds(right * CH, CH), cs] = acc
            ccs = pl.ds(HC + sub * SB, SB)
            acc = out_ref[pl.ds(left * CH, CH), ccs] \
                + ccw_comm[2, :, pl.ds(sub * SB, SB)]
            out_ref[pl.ds(left * CH, CH), ccs] = acc
            cw, ccw = _ag(0, sub)
            cw.start()
            ccw.start()
        for s in (1, 2):
            for sub in range(NSUB):
                cw, ccw = _ag(s - 1, sub)
                cw.wait()
                ccw.wait()
                cw, ccw = _ag(s, sub)
                cw.start()
                ccw.start()
        for sub in range(NSUB):
            cw, ccw = _ag(2, sub)
            cw.wait()
            ccw.wait()


def kernel(x, Wq, K_ext, V_ext, Wo):
    i = lax.axis_index("i")
    x2 = x.reshape(SQ, DIN)
    K2 = K_ext.reshape(SKV, 32, DH)
    V2 = V_ext.reshape(SKV, 32, DH)
    iarr = jnp.full((1,), i, jnp.int32)

    grid_spec = pltpu.PrefetchScalarGridSpec(
        num_scalar_prefetch=1,
        grid=(N_DEV,),
        in_specs=[
            pl.BlockSpec((BQ, DIN), lambda qc, ir: (_gc_of(qc, ir[0]), 0)),
            pl.BlockSpec((DIN, DSH), lambda qc, ir: (0, 0)),
            pl.BlockSpec(memory_space=pl.ANY),
            pl.BlockSpec(memory_space=pl.ANY),
            pl.BlockSpec((DSH, DIN), lambda qc, ir: (0, 0)),
        ],
        out_specs=pl.BlockSpec((SQ, DIN), lambda qc, ir: (0, 0)),
        scratch_shapes=[
            pltpu.VMEM((BQ, DSH), jnp.float32),
            pltpu.VMEM((HPD, SKV, DH), jnp.float32),
            pltpu.VMEM((HPD, SKV, DH), jnp.float32),
            pltpu.SemaphoreType.DMA((2, HPD)),
            pltpu.VMEM((N_DEV - 1, CH, HC), jnp.float32),
            pltpu.VMEM((N_DEV - 1, CH, HC), jnp.float32),
        ] + [pltpu.SemaphoreType.DMA((N_DEV - 1, NSUB))] * 8,
    )
    out = pl.pallas_call(
        _body,
        grid_spec=grid_spec,
        out_shape=jax.ShapeDtypeStruct((SQ, DIN), jnp.float32),
        compiler_params=pltpu.CompilerParams(
            dimension_semantics=("arbitrary",),
            collective_id=0,
        ),
    )(iarr, x2, Wq, K2, V2, Wo)
    return out.reshape(1, SQ, DIN)
